# baseline (speedup 1.0000x reference)
import functools

import numpy as np
import jax
import jax.numpy as jnp
from jax import lax
from jax.experimental import pallas as pl
from jax.experimental.pallas import tpu as pltpu
from jax.experimental.pallas import tpu_sc as plsc

_NBX = 512
_NBY = 512
_MOV_RHS = 800000
_UX = 1.0 / _NBX
_UY = 1.0 / _NBY
_MAPW = _NBX * _NBY
_NN = 1000000
_NW = 32
_PER_W = 31232
_CHUNK = 2048
_TAIL = _PER_W - 15 * _CHUNK
_EXTRA = _NN - _NW * _PER_W
_EXTRA_OFF = _NW * _PER_W
_CHUNK_SIZES = [_CHUNK] * 15 + [_TAIL]
_NROW = (_CHUNK * 4) // 128


def _np_dct2(n):
    i = np.arange(n)
    k = i.reshape(-1, 1)
    return np.cos(np.pi * (i + 0.5) * k / n).astype(np.float32)


def _np_idct(n):
    i = np.arange(n).reshape(-1, 1)
    k = np.arange(n)
    m = np.cos(np.pi * (i + 0.5) * k / n)
    w = np.full(n, 2.0 / n)
    w[0] = 1.0 / n
    return (m * w).astype(np.float32)


def _np_idxst(n):
    i = np.arange(n).reshape(-1, 1)
    k = np.arange(n)
    m = np.sin(np.pi * (i + 0.5) * k / n)
    w = np.full(n, 2.0 / n)
    w[0] = 1.0 / n
    return (m * w).astype(np.float32)


def _np_fft_scale():
    w_j = (np.arange(_NBX) * (2.0 * np.pi / _NBX)).reshape(_NBX, 1)
    w_k = (np.arange(_NBY) * (2.0 * np.pi / _NBY)).reshape(1, _NBY)
    w_k = w_k * (_UX / _UY)
    s = w_j ** 2 + w_k ** 2
    s[0, 0] = 1.0
    pot = 1.0 / s
    pot[0, 0] = 0.0
    return (pot.astype(np.float32),
            (w_j * pot * 0.5).astype(np.float32),
            (w_k * pot * 0.5).astype(np.float32))


_CXn = _np_dct2(_NBX)
_CYTn = _np_dct2(_NBY).T.copy()
_IXn = _np_idct(_NBX)
_IYTn = _np_idct(_NBY).T.copy()
_SXn = _np_idxst(_NBX)
_SYTn = _np_idxst(_NBY).T.copy()
_PSn, _FXSn, _FYSn = _np_fft_scale()


_mesh = plsc.VectorSubcoreMesh(core_axis_name="c", subcore_axis_name="s")


@functools.partial(
    pl.kernel,
    mesh=_mesh,
    compiler_params=pltpu.CompilerParams(needs_layout_passes=False),
    out_type=jax.ShapeDtypeStruct((2, 2 * _MAPW), jnp.float32),
    scratch_types=[
        pltpu.VMEM((2 * _CHUNK,), jnp.float32),
        pltpu.VMEM((2 * _CHUNK,), jnp.float32),
        pltpu.VMEM((2 * _CHUNK,), jnp.float32),
        pltpu.VMEM((2 * _CHUNK,), jnp.float32),
        pltpu.VMEM((_CHUNK,), jnp.float32),
        pltpu.VMEM((_CHUNK,), jnp.float32),
        pltpu.VMEM((_CHUNK,), jnp.float32),
        pltpu.VMEM((_CHUNK,), jnp.float32),
        pltpu.VMEM((_NROW, 128), jnp.int32),
        pltpu.VMEM((_NROW, 128), jnp.int32),
        pltpu.VMEM((_NROW, 128), jnp.float32),
        pltpu.VMEM((_NROW, 128), jnp.float32),
        pltpu.VMEM((4096,), jnp.float32),
        pltpu.VMEM_SHARED((2 * _MAPW,), jnp.float32),
        pltpu.SemaphoreType.DMA,
        pltpu.SemaphoreType.DMA,
    ],
)
def _sc_scatter(posh, sizeh, wh, eh, out,
                posb0, posb1, sizeb0, sizeb1,
                wb0, wb1, eb0, eb1,
                idxb0, idxb1, valb0, valb1, zbuf, shared,
                in_sem, sc_sem):
    cid = lax.axis_index("c")
    sid = lax.axis_index("s")
    wid = cid * 16 + sid
    base = wid * _PER_W
    posbs = (posb0, posb1)
    sizebs = (sizeb0, sizeb1)
    wbs = (wb0, wb1)
    ebs = (eb0, eb1)
    idxbs = (idxb0, idxb1)
    valbs = (valb0, valb1)

    ev2 = 2 * lax.broadcasted_iota(jnp.int32, (16,), 0)

    def _issue_loads(cb, p, n):
        return [
            pltpu.async_copy(posh.at[pl.ds(2 * cb, 2 * n)],
                             posbs[p].at[pl.ds(0, 2 * n)], in_sem),
            pltpu.async_copy(sizeh.at[pl.ds(2 * cb, 2 * n)],
                             sizebs[p].at[pl.ds(0, 2 * n)], in_sem),
            pltpu.async_copy(wh.at[pl.ds(cb, n)],
                             wbs[p].at[pl.ds(0, n)], in_sem),
            pltpu.async_copy(eh.at[pl.ds(cb, n)],
                             ebs[p].at[pl.ds(0, n)], in_sem),
        ]

    handles = _issue_loads(base, 0, _CHUNK_SIZES[0])

    def _zero(i, carry):
        zbuf[pl.ds(i * 16, 16)] = jnp.zeros((16,), jnp.float32)
        return carry

    lax.fori_loop(0, 256, _zero, 0)
    for k in range(8):
        pltpu.sync_copy(zbuf, shared.at[pl.ds(sid * 32768 + k * 4096, 4096)])
    plsc.subcore_barrier()

    def _compute(cb, p, iters):
        posb, sizeb, wb, eb = posbs[p], sizebs[p], wbs[p], ebs[p]
        idxb, valb = idxbs[p], valbs[p]

        def _iter(i, carry2):
            o = i * 16
            ie = 2 * o + ev2
            px = plsc.load_gather(posb, [ie])
            py = plsc.load_gather(posb, [ie + 1])
            sx = plsc.load_gather(sizeb, [ie])
            sy = plsc.load_gather(sizeb, [ie + 1])
            wv = wb[pl.ds(o, 16)]
            ev = eb[pl.ds(o, 16)]
            amt = wv * ev * sx * sy * jnp.float32(_MAPW)
            xs = px * jnp.float32(_NBX)
            ys = py * jnp.float32(_NBY)
            ix0 = jnp.clip(xs.astype(jnp.int32), 0, _NBX - 1)
            iy0 = jnp.clip(ys.astype(jnp.int32), 0, _NBY - 1)
            fx = jnp.clip(xs - ix0.astype(jnp.float32), 0.0, 1.0)
            fy = jnp.clip(ys - iy0.astype(jnp.float32), 0.0, 1.0)
            ix1 = jnp.minimum(ix0 + 1, _NBX - 1)
            iy1 = jnp.minimum(iy0 + 1, _NBY - 1)
            gid = cb + o + lax.broadcasted_iota(jnp.int32, (16,), 0)
            selo = jnp.where(gid >= _MOV_RHS, jnp.int32(_MAPW), jnp.int32(0))
            b0 = selo + ix0 * _NBY
            b1 = selo + ix1 * _NBY
            ax = amt * fx
            gx = amt - ax
            v01 = gx * fy
            v00 = gx - v01
            v11 = ax * fy
            v10 = ax - v11
            j = i // 2
            col = (i % 2) * 64
            idxb[j, pl.ds(col, 16)] = b0 + iy0
            idxb[j, pl.ds(col + 16, 16)] = b1 + iy0
            idxb[j, pl.ds(col + 32, 16)] = b0 + iy1
            idxb[j, pl.ds(col + 48, 16)] = b1 + iy1
            valb[j, pl.ds(col, 16)] = v00
            valb[j, pl.ds(col + 16, 16)] = v10
            valb[j, pl.ds(col + 32, 16)] = v01
            valb[j, pl.ds(col + 48, 16)] = v11
            return carry2

        lax.fori_loop(0, iters, _iter, 0)

    def _fire(p, rows):
        def _f(j, carry):
            pltpu.async_copy(valbs[p].at[j], shared.at[idxbs[p].at[j]],
                             sc_sem, add=True)
            return carry
        lax.fori_loop(0, rows, _f, 0)

    def _drain(p, rows):
        def _d(j, carry):
            pltpu.make_async_copy(valbs[p].at[j], shared.at[idxbs[p].at[j]],
                                  sc_sem).wait()
            return carry
        lax.fori_loop(0, rows, _d, 0)

    nchunk = len(_CHUNK_SIZES)
    for c, size in enumerate(_CHUNK_SIZES):
        p = c & 1
        for h in handles:
            h.wait()
        if c + 1 < nchunk:
            handles = _issue_loads(base + (c + 1) * _CHUNK, 1 - p,
                                   _CHUNK_SIZES[c + 1])
        if c >= 2:
            _drain(p, _CHUNK_SIZES[c - 2] // 32)
        _compute(base + c * _CHUNK, p, size // 16)
        _fire(p, size // 32)

    _drain(nchunk % 2, _CHUNK_SIZES[nchunk - 2] // 32)
    _drain(1 - nchunk % 2, _CHUNK_SIZES[nchunk - 1] // 32)

    @pl.when(wid == 0)
    def _extra():
        pltpu.sync_copy(posh.at[pl.ds(2 * _EXTRA_OFF, 2 * _EXTRA)],
                        posb0.at[pl.ds(0, 2 * _EXTRA)])
        pltpu.sync_copy(sizeh.at[pl.ds(2 * _EXTRA_OFF, 2 * _EXTRA)],
                        sizeb0.at[pl.ds(0, 2 * _EXTRA)])
        pltpu.sync_copy(wh.at[pl.ds(_EXTRA_OFF, _EXTRA)],
                        wb0.at[pl.ds(0, _EXTRA)])
        pltpu.sync_copy(eh.at[pl.ds(_EXTRA_OFF, _EXTRA)],
                        eb0.at[pl.ds(0, _EXTRA)])
        _compute(_EXTRA_OFF, 0, _EXTRA // 16)
        _fire(0, _EXTRA // 32)
        _drain(0, _EXTRA // 32)

    plsc.subcore_barrier()
    pltpu.sync_copy(shared.at[pl.ds(sid * 32768, 32768)],
                    out.at[cid, pl.ds(sid * 32768, 32768)])


def _tc_body(parts, init, cx, cyt, ixm, iyt, sxm, syt, ps, fxs, fys,
             en_ref, ov_ref, grad_ref):
    mov = parts[0, 0] + parts[1, 0] + init[...]
    fil = parts[0, 1] + parts[1, 1]
    dmap = mov + fil
    ov = jnp.sum(jnp.maximum(mov - 1.0, 0.0)) * jnp.float32(_UX * _UY)
    ov_ref[...] = jnp.full((1, 1), ov, jnp.float32)

    def mm(a, b):
        return lax.dot_general(a, b, (((1,), (0,)), ((), ())),
                               preferred_element_type=jnp.float32)

    co = mm(mm(cx[...], dmap), cyt[...])
    fxm = mm(mm(sxm[...], co * fxs[...]), iyt[...])
    fym = mm(mm(ixm[...], co * fys[...]), syt[...])
    pot = mm(mm(ixm[...], co * ps[...]), iyt[...])
    en_ref[...] = jnp.full((1, 1), jnp.sum(pot * dmap), jnp.float32)
    grad_ref[0] = fxm
    grad_ref[1] = fym


_tc = pl.pallas_call(
    _tc_body,
    out_shape=(
        jax.ShapeDtypeStruct((1, 1), jnp.float32),
        jax.ShapeDtypeStruct((1, 1), jnp.float32),
        jax.ShapeDtypeStruct((2, _NBX, _NBY), jnp.float32),
    ),
)


def kernel(node_pos, node_size, node_weight, expand_ratio, init_density_map):
    parts = _sc_scatter(node_pos.reshape(-1), node_size.reshape(-1),
                        node_weight, expand_ratio)
    parts4 = parts.reshape(2, 2, _NBX, _NBY)
    en, ov, grad = _tc(parts4, init_density_map,
                       jnp.asarray(_CXn), jnp.asarray(_CYTn),
                       jnp.asarray(_IXn), jnp.asarray(_IYTn),
                       jnp.asarray(_SXn), jnp.asarray(_SYTn),
                       jnp.asarray(_PSn), jnp.asarray(_FXSn),
                       jnp.asarray(_FYSn))
    return en[0, 0], ov[0, 0], grad


# speedup vs baseline: 19.0863x; 19.0863x over previous
import functools

import numpy as np
import jax
import jax.numpy as jnp
from jax import lax
from jax.experimental import pallas as pl
from jax.experimental.pallas import tpu as pltpu
from jax.experimental.pallas import tpu_sc as plsc

_NBX = 512
_NBY = 512
_MOV_RHS = 800000
_UX = 1.0 / _NBX
_UY = 1.0 / _NBY
_MAPW = _NBX * _NBY
_NN = 1000000
_NW = 32
_PER_W = 31232
_CHUNK = 2048
_TAIL = _PER_W - 15 * _CHUNK
_EXTRA = _NN - _NW * _PER_W
_EXTRA_OFF = _NW * _PER_W
_CHUNK_SIZES = [_CHUNK] * 15 + [_TAIL]
_NROW = (_CHUNK * 4) // 128


def _np_dct2(n):
    i = np.arange(n)
    k = i.reshape(-1, 1)
    return np.cos(np.pi * (i + 0.5) * k / n).astype(np.float32)


def _np_idct(n):
    i = np.arange(n).reshape(-1, 1)
    k = np.arange(n)
    m = np.cos(np.pi * (i + 0.5) * k / n)
    w = np.full(n, 2.0 / n)
    w[0] = 1.0 / n
    return (m * w).astype(np.float32)


def _np_idxst(n):
    i = np.arange(n).reshape(-1, 1)
    k = np.arange(n)
    m = np.sin(np.pi * (i + 0.5) * k / n)
    w = np.full(n, 2.0 / n)
    w[0] = 1.0 / n
    return (m * w).astype(np.float32)


def _np_fft_scale():
    w_j = (np.arange(_NBX) * (2.0 * np.pi / _NBX)).reshape(_NBX, 1)
    w_k = (np.arange(_NBY) * (2.0 * np.pi / _NBY)).reshape(1, _NBY)
    w_k = w_k * (_UX / _UY)
    s = w_j ** 2 + w_k ** 2
    s[0, 0] = 1.0
    pot = 1.0 / s
    pot[0, 0] = 0.0
    return (pot.astype(np.float32),
            (w_j * pot * 0.5).astype(np.float32),
            (w_k * pot * 0.5).astype(np.float32))


_CXn = _np_dct2(_NBX)
_CYTn = _np_dct2(_NBY).T.copy()
_IXn = _np_idct(_NBX)
_IYTn = _np_idct(_NBY).T.copy()
_SXn = _np_idxst(_NBX)
_SYTn = _np_idxst(_NBY).T.copy()
_PSn, _FXSn, _FYSn = _np_fft_scale()


_mesh = plsc.VectorSubcoreMesh(core_axis_name="c", subcore_axis_name="s")


@functools.partial(
    pl.kernel,
    mesh=_mesh,
    out_type=jax.ShapeDtypeStruct((2, 2 * _MAPW), jnp.float32),
    scratch_types=[
        pltpu.VMEM((_CHUNK,), jnp.float32),
        pltpu.VMEM((_CHUNK,), jnp.float32),
        pltpu.VMEM((_CHUNK,), jnp.float32),
        pltpu.VMEM((_CHUNK,), jnp.float32),
        pltpu.VMEM((_CHUNK,), jnp.float32),
        pltpu.VMEM((_CHUNK,), jnp.float32),
        pltpu.VMEM((_CHUNK,), jnp.float32),
        pltpu.VMEM((_CHUNK,), jnp.float32),
        pltpu.VMEM((_CHUNK,), jnp.float32),
        pltpu.VMEM((_CHUNK,), jnp.float32),
        pltpu.VMEM((_CHUNK,), jnp.float32),
        pltpu.VMEM((_CHUNK,), jnp.float32),
        pltpu.VMEM((_NROW, 128), jnp.int32),
        pltpu.VMEM((_NROW, 128), jnp.int32),
        pltpu.VMEM((_NROW, 128), jnp.float32),
        pltpu.VMEM((_NROW, 128), jnp.float32),
        pltpu.VMEM((4096,), jnp.float32),
        pltpu.VMEM_SHARED((2 * _MAPW,), jnp.float32),
        pltpu.SemaphoreType.DMA,
        pltpu.SemaphoreType.DMA,
    ],
)
def _sc_scatter(posT, sizeT, wh, eh, out,
                xb0, xb1, yb0, yb1, sxb0, sxb1, syb0, syb1,
                wb0, wb1, eb0, eb1,
                idxb0, idxb1, valb0, valb1, zbuf, shared,
                in_sem, sc_sem):
    cid = lax.axis_index("c")
    sid = lax.axis_index("s")
    wid = cid * 16 + sid
    base = wid * _PER_W
    xbs = (xb0, xb1)
    ybs = (yb0, yb1)
    sxbs = (sxb0, sxb1)
    sybs = (syb0, syb1)
    wbs = (wb0, wb1)
    ebs = (eb0, eb1)
    idxbs = (idxb0, idxb1)
    valbs = (valb0, valb1)

    def _issue_loads(cb, p, n):
        return [
            pltpu.async_copy(posT.at[pl.ds(cb, n)],
                             xbs[p].at[pl.ds(0, n)], in_sem),
            pltpu.async_copy(posT.at[pl.ds(_NN + cb, n)],
                             ybs[p].at[pl.ds(0, n)], in_sem),
            pltpu.async_copy(sizeT.at[pl.ds(cb, n)],
                             sxbs[p].at[pl.ds(0, n)], in_sem),
            pltpu.async_copy(sizeT.at[pl.ds(_NN + cb, n)],
                             sybs[p].at[pl.ds(0, n)], in_sem),
            pltpu.async_copy(wh.at[pl.ds(cb, n)],
                             wbs[p].at[pl.ds(0, n)], in_sem),
            pltpu.async_copy(eh.at[pl.ds(cb, n)],
                             ebs[p].at[pl.ds(0, n)], in_sem),
        ]

    handles = _issue_loads(base, 0, _CHUNK_SIZES[0])

    def _zero(i, carry):
        zbuf[pl.ds(i * 16, 16)] = jnp.zeros((16,), jnp.float32)
        return carry

    lax.fori_loop(0, 256, _zero, 0)
    for k in range(8):
        pltpu.sync_copy(zbuf, shared.at[pl.ds(sid * 32768 + k * 4096, 4096)])
    plsc.subcore_barrier()

    def _compute(cb, p, iters):
        xb, yb, sxb, syb, wb, eb = (xbs[p], ybs[p], sxbs[p], sybs[p],
                                    wbs[p], ebs[p])
        idxb, valb = idxbs[p], valbs[p]

        def _iter(i, carry2):
            o = i * 16
            px = xb[pl.ds(o, 16)]
            py = yb[pl.ds(o, 16)]
            sx = sxb[pl.ds(o, 16)]
            sy = syb[pl.ds(o, 16)]
            wv = wb[pl.ds(o, 16)]
            ev = eb[pl.ds(o, 16)]
            amt = wv * ev * sx * sy * jnp.float32(_MAPW)
            xs = px * jnp.float32(_NBX)
            ys = py * jnp.float32(_NBY)
            ix0 = jnp.clip(xs.astype(jnp.int32), 0, _NBX - 1)
            iy0 = jnp.clip(ys.astype(jnp.int32), 0, _NBY - 1)
            fx = jnp.clip(xs - ix0.astype(jnp.float32), 0.0, 1.0)
            fy = jnp.clip(ys - iy0.astype(jnp.float32), 0.0, 1.0)
            ix1 = jnp.minimum(ix0 + 1, _NBX - 1)
            iy1 = jnp.minimum(iy0 + 1, _NBY - 1)
            gid = cb + o + lax.broadcasted_iota(jnp.int32, (16,), 0)
            selo = jnp.where(gid >= _MOV_RHS, jnp.int32(_MAPW), jnp.int32(0))
            b0 = selo + ix0 * _NBY
            b1 = selo + ix1 * _NBY
            ax = amt * fx
            gx = amt - ax
            v01 = gx * fy
            v00 = gx - v01
            v11 = ax * fy
            v10 = ax - v11
            j = i // 2
            col = (i % 2) * 64
            idxb[j, pl.ds(col, 16)] = b0 + iy0
            idxb[j, pl.ds(col + 16, 16)] = b1 + iy0
            idxb[j, pl.ds(col + 32, 16)] = b0 + iy1
            idxb[j, pl.ds(col + 48, 16)] = b1 + iy1
            valb[j, pl.ds(col, 16)] = v00
            valb[j, pl.ds(col + 16, 16)] = v10
            valb[j, pl.ds(col + 32, 16)] = v01
            valb[j, pl.ds(col + 48, 16)] = v11
            return carry2

        lax.fori_loop(0, iters, _iter, 0)

    def _fire(p, rows):
        def _f(j, carry):
            pltpu.async_copy(valbs[p].at[j], shared.at[idxbs[p].at[j]],
                             sc_sem, add=True)
            return carry
        lax.fori_loop(0, rows, _f, 0)

    def _drain(p, rows):
        def _d(j, carry):
            pltpu.make_async_copy(valbs[p].at[j], shared.at[idxbs[p].at[j]],
                                  sc_sem).wait()
            return carry
        lax.fori_loop(0, rows, _d, 0)

    nchunk = len(_CHUNK_SIZES)
    for c, size in enumerate(_CHUNK_SIZES):
        p = c & 1
        for h in handles:
            h.wait()
        if c + 1 < nchunk:
            handles = _issue_loads(base + (c + 1) * _CHUNK, 1 - p,
                                   _CHUNK_SIZES[c + 1])
        if c >= 2:
            _drain(p, _CHUNK_SIZES[c - 2] // 32)
        _compute(base + c * _CHUNK, p, size // 16)
        _fire(p, size // 32)

    _drain(nchunk % 2, _CHUNK_SIZES[nchunk - 2] // 32)
    _drain(1 - nchunk % 2, _CHUNK_SIZES[nchunk - 1] // 32)

    @pl.when(wid == 0)
    def _extra():
        for off, b in ((_EXTRA_OFF, xb0), (_NN + _EXTRA_OFF, yb0)):
            pltpu.sync_copy(posT.at[pl.ds(off, _EXTRA)],
                            b.at[pl.ds(0, _EXTRA)])
        for off, b in ((_EXTRA_OFF, sxb0), (_NN + _EXTRA_OFF, syb0)):
            pltpu.sync_copy(sizeT.at[pl.ds(off, _EXTRA)],
                            b.at[pl.ds(0, _EXTRA)])
        pltpu.sync_copy(wh.at[pl.ds(_EXTRA_OFF, _EXTRA)],
                        wb0.at[pl.ds(0, _EXTRA)])
        pltpu.sync_copy(eh.at[pl.ds(_EXTRA_OFF, _EXTRA)],
                        eb0.at[pl.ds(0, _EXTRA)])
        _compute(_EXTRA_OFF, 0, _EXTRA // 16)
        _fire(0, _EXTRA // 32)
        _drain(0, _EXTRA // 32)

    plsc.subcore_barrier()
    pltpu.sync_copy(shared.at[pl.ds(sid * 32768, 32768)],
                    out.at[cid, pl.ds(sid * 32768, 32768)])


def _tc_body(parts, init, cx, cyt, ixm, iyt, sxm, syt, ps, fxs, fys,
             en_ref, ov_ref, grad_ref):
    mov = parts[0, 0] + parts[1, 0] + init[...]
    fil = parts[0, 1] + parts[1, 1]
    dmap = mov + fil
    ov = jnp.sum(jnp.maximum(mov - 1.0, 0.0)) * jnp.float32(_UX * _UY)
    ov_ref[...] = jnp.full((1, 1), ov, jnp.float32)

    def mm(a, b):
        return lax.dot_general(a, b, (((1,), (0,)), ((), ())),
                               preferred_element_type=jnp.float32)

    co = mm(mm(cx[...], dmap), cyt[...])
    fxm = mm(mm(sxm[...], co * fxs[...]), iyt[...])
    fym = mm(mm(ixm[...], co * fys[...]), syt[...])
    pot = mm(mm(ixm[...], co * ps[...]), iyt[...])
    en_ref[...] = jnp.full((1, 1), jnp.sum(pot * dmap), jnp.float32)
    grad_ref[0] = fxm
    grad_ref[1] = fym


_tc = pl.pallas_call(
    _tc_body,
    out_shape=(
        jax.ShapeDtypeStruct((1, 1), jnp.float32),
        jax.ShapeDtypeStruct((1, 1), jnp.float32),
        jax.ShapeDtypeStruct((2, _NBX, _NBY), jnp.float32),
    ),
)


def kernel(node_pos, node_size, node_weight, expand_ratio, init_density_map):
    parts = _sc_scatter(node_pos.T.reshape(-1), node_size.T.reshape(-1),
                        node_weight, expand_ratio)
    parts4 = parts.reshape(2, 2, _NBX, _NBY)
    en, ov, grad = _tc(parts4, init_density_map,
                       jnp.asarray(_CXn), jnp.asarray(_CYTn),
                       jnp.asarray(_IXn), jnp.asarray(_IYTn),
                       jnp.asarray(_SXn), jnp.asarray(_SYTn),
                       jnp.asarray(_PSn), jnp.asarray(_FXSn),
                       jnp.asarray(_FYSn))
    return en[0, 0], ov[0, 0], grad
